# baseline (speedup 1.0000x reference)
import functools

import numpy as np
import jax
import jax.numpy as jnp
from jax import lax
from jax.experimental import pallas as pl
from jax.experimental.pallas import tpu as pltpu
from jax.experimental.pallas import tpu_sc as plsc

_VOCAB = 1000000
_EMBED = 32
_HALF = 16
_B = 4
_S = 4096
_NC = 2
_NS = 16
_NW = _NC * _NS
_ROWS = _B * _S
_RPW = _ROWS // _NW
_SPW = _RPW // _B
_CHUNK = 128
_NCHUNK = _RPW // _CHUNK
_QP = _CHUNK // _B

_theta32 = (1.0 / (10000.0 ** (np.arange(_HALF, dtype=np.float32) / np.float32(_HALF)))).astype(np.float32)
_ang32 = (np.arange(_S, dtype=np.float32)[:, None] * _theta32[None, :]).astype(np.float32)
_COS_TABLE = np.cos(_ang32.astype(np.float64)).astype(np.float32).reshape(-1)
_SIN_TABLE = np.sin(_ang32.astype(np.float64)).astype(np.float32).reshape(-1)

_mesh = plsc.VectorSubcoreMesh(core_axis_name="c", subcore_axis_name="s")


@functools.partial(
    pl.kernel,
    mesh=_mesh,
    compiler_params=pltpu.CompilerParams(use_tc_tiling_on_sc=False),
    out_type=jax.ShapeDtypeStruct((_ROWS, _EMBED), jnp.float32),
    scratch_types=[
        pltpu.VMEM((_NCHUNK, _CHUNK), jnp.int32),
        pltpu.VMEM((2, _CHUNK, _EMBED), jnp.float32),
        pltpu.VMEM((2, _CHUNK, _EMBED), jnp.float32),
        pltpu.VMEM((_SPW * _HALF,), jnp.float32),
        pltpu.VMEM((_SPW * _HALF,), jnp.float32),
        pltpu.SemaphoreType.DMA,
        pltpu.SemaphoreType.DMA,
        pltpu.SemaphoreType.DMA,
        pltpu.SemaphoreType.DMA,
    ],
)
def _embed_rotary(idx_hbm, table_hbm, cos_hbm, sin_hbm, out_hbm,
                  idx_v, rows_v, out_v, cos_v, sin_v,
                  gsem0, gsem1, osem0, osem1):
    wid = lax.axis_index("s") * _NC + lax.axis_index("c")
    base = wid * _RPW
    pbase = wid * _SPW * _HALF

    for c in range(_NCHUNK):
        pltpu.sync_copy(idx_hbm.at[pl.ds(base + c * _CHUNK, _CHUNK)], idx_v.at[c])
    pltpu.sync_copy(cos_hbm.at[pl.ds(pbase, _SPW * _HALF)], cos_v)
    pltpu.sync_copy(sin_hbm.at[pl.ds(pbase, _SPW * _HALF)], sin_v)

    gsems = (gsem0, gsem1)
    osems = (osem0, osem1)

    def fire(c):
        buf = c % 2
        return pltpu.async_copy(
            table_hbm.at[idx_v.at[c]],
            rows_v.at[buf],
            gsems[buf],
        )

    handle = {0: fire(0), 1: None}
    ohandle = {0: None, 1: None}
    for c in range(_NCHUNK):
        buf = c % 2
        if c + 1 < _NCHUNK:
            handle[(c + 1) % 2] = fire(c + 1)
        handle[buf].wait()
        if ohandle[buf] is not None:
            ohandle[buf].wait()

        def body(q, carry, c=c, buf=buf):
            cos = cos_v[pl.ds((c * _QP + q) * _HALF, _HALF)]
            sin = sin_v[pl.ds((c * _QP + q) * _HALF, _HALF)]
            for b in range(_B):
                n = q * _B + b
                x1 = rows_v[buf, n, 0:_HALF]
                x2 = rows_v[buf, n, _HALF:_EMBED]
                out_v[buf, n, 0:_HALF] = x1 * cos - x2 * sin
                out_v[buf, n, _HALF:_EMBED] = x1 * sin + x2 * cos
            return carry

        lax.fori_loop(0, _QP, body, 0, unroll=4)
        ohandle[buf] = pltpu.async_copy(
            out_v.at[buf],
            out_hbm.at[pl.ds(base + c * _CHUNK, _CHUNK)],
            osems[buf],
        )
    ohandle[0].wait()
    ohandle[1].wait()


def kernel(x, token_embedding):
    xt = x.T.reshape(-1)
    out = _embed_rotary(xt, token_embedding,
                        jnp.asarray(_COS_TABLE), jnp.asarray(_SIN_TABLE))
    return out.reshape(_S, _B, _EMBED)
